# baseline (speedup 1.0000x reference)
import functools

import jax
import jax.numpy as jnp
from jax import lax
from jax.experimental import pallas as pl
from jax.experimental.pallas import tpu as pltpu
from jax.experimental.pallas import tpu_sc as plsc

NC = 2
NS = 16
L = 16
NW = NC * NS

_mesh = lambda: plsc.VectorSubcoreMesh(
    core_axis_name="c", subcore_axis_name="s", num_cores=NC, num_subcores=NS
)


def _zero_vmem_2d(ref, rows, width):
    z = jnp.zeros((L,), jnp.float32)

    def body(i, _):
        for j in range(width // L):
            ref[i, pl.ds(j * L, L)] = z
        return 0

    lax.fori_loop(0, rows, body, 0)


def _zero_vmem_1d(ref, n):
    z = jnp.zeros((L,), jnp.float32)

    def body(i, _):
        ref[pl.ds(i * L, L)] = z
        return 0

    lax.fori_loop(0, n // L, body, 0)


def _make_degree_kernel(E, NP, K):
    e_per_tile = E // NW
    n_chunks = e_per_tile // K
    npt = NP // NS

    @functools.partial(
        pl.kernel,
        out_type=jax.ShapeDtypeStruct((NC, 2, NP), jnp.float32),
        mesh=_mesh(),
        scratch_types=[
            pltpu.VMEM_SHARED((NP,), jnp.float32),
            pltpu.VMEM_SHARED((NP,), jnp.float32),
            pltpu.VMEM((K,), jnp.int32),
            pltpu.VMEM((K,), jnp.int32),
            pltpu.VMEM((K,), jnp.float32),
            pltpu.VMEM((npt,), jnp.float32),
        ],
    )
    def deg_kernel(src_hbm, dst_hbm, out_hbm, deg_o, deg_i, sidx, didx, ones, zbuf):
        c = lax.axis_index("c")
        s = lax.axis_index("s")
        g = c * NS + s

        one = jnp.ones((L,), jnp.float32)

        def ones_body(i, _):
            ones[pl.ds(i * L, L)] = one
            return 0

        lax.fori_loop(0, K // L, ones_body, 0)
        _zero_vmem_1d(zbuf, npt)

        base_n = s * npt
        pltpu.sync_copy(zbuf, deg_o.at[pl.ds(base_n, npt)])
        pltpu.sync_copy(zbuf, deg_i.at[pl.ds(base_n, npt)])
        plsc.subcore_barrier()

        def chunk(i, _):
            base_e = g * e_per_tile + i * K
            pltpu.sync_copy(src_hbm.at[pl.ds(base_e, K)], sidx)
            pltpu.sync_copy(dst_hbm.at[pl.ds(base_e, K)], didx)
            pltpu.sync_copy(ones, deg_o.at[sidx], add=True)
            pltpu.sync_copy(ones, deg_i.at[didx], add=True)
            return 0

        lax.fori_loop(0, n_chunks, chunk, 0)
        plsc.subcore_barrier()

        pltpu.sync_copy(deg_o.at[pl.ds(base_n, npt)], out_hbm.at[c, 0, pl.ds(base_n, npt)])
        pltpu.sync_copy(deg_i.at[pl.ds(base_n, npt)], out_hbm.at[c, 1, pl.ds(base_n, npt)])

    return deg_kernel


def _make_agg_kernel(NP, D, E, K):
    U = 5
    e_per_tile = E // NW
    n_chunks = e_per_tile // K
    assert n_chunks % U == 0
    rpt = NP // NS
    ZR = 64
    assert rpt % ZR == 0 and rpt % 8 == 0

    @functools.partial(
        pl.kernel,
        out_type=jax.ShapeDtypeStruct((NC, NP, D), jnp.float32),
        mesh=_mesh(),
        scratch_types=[
            pltpu.VMEM_SHARED((NP, D), jnp.float32),
            pltpu.VMEM((U, K), jnp.int32),
            pltpu.VMEM((U, K), jnp.int32),
            pltpu.VMEM((U, K, D), jnp.float32),
            pltpu.VMEM((ZR, D), jnp.float32),
            pltpu.SemaphoreType.DMA,
            pltpu.SemaphoreType.DMA,
        ],
    )
    def agg_kernel(h_hbm, src_hbm, dst_hbm, out_hbm, agg, sidx, didx, rows, zbuf,
                   gsem, ssem):
        c = lax.axis_index("c")
        s = lax.axis_index("s")
        g = c * NS + s

        _zero_vmem_2d(zbuf, ZR, D)
        row0 = s * rpt
        for r in range(rpt // ZR):
            pltpu.sync_copy(zbuf, agg.at[pl.ds(row0 + r * ZR, ZR)])
        plsc.subcore_barrier()

        def block(j, _):
            gd = []
            for u in range(U):
                base_e = g * e_per_tile + (j * U + u) * K
                pltpu.sync_copy(src_hbm.at[pl.ds(base_e, K)], sidx.at[u])
                pltpu.sync_copy(dst_hbm.at[pl.ds(base_e, K)], didx.at[u])
                gd.append(pltpu.async_copy(h_hbm.at[sidx.at[u]], rows.at[u], gsem))
            sd = []
            for u in range(U):
                gd[u].wait()
                sd.append(pltpu.async_copy(rows.at[u], agg.at[didx.at[u]], ssem,
                                           add=True))
            for d in sd:
                d.wait()
            return 0

        lax.fori_loop(0, n_chunks // U, block, 0)
        plsc.subcore_barrier()

        pltpu.sync_copy(agg.at[pl.ds(row0, rpt)], out_hbm.at[c, pl.ds(row0, rpt)])

    return agg_kernel


def _tc_mm_scale(x, w, ns):
    def body(x_ref, w_ref, ns_ref, o_ref):
        o_ref[...] = (
            jnp.dot(x_ref[...], w_ref[...], preferred_element_type=jnp.float32)
            * ns_ref[...]
        )

    return pl.pallas_call(
        body,
        out_shape=jax.ShapeDtypeStruct((x.shape[0], w.shape[1]), jnp.float32),
    )(x, w, ns)


def _tc_layer_mid(aggp, nd, b1, w, ns):
    N = nd.shape[0]

    def body(ap_ref, nd_ref, b1_ref, w_ref, ns_ref, o_ref):
        a = ap_ref[0, :N] + ap_ref[1, :N]
        h = jnp.maximum(a * nd_ref[...] + b1_ref[...], 0.0)
        o_ref[...] = (
            jnp.dot(h, w_ref[...], preferred_element_type=jnp.float32) * ns_ref[...]
        )

    return pl.pallas_call(
        body,
        out_shape=jax.ShapeDtypeStruct((N, w.shape[1]), jnp.float32),
    )(aggp, nd, b1, w, ns)


def _tc_final(aggp, nd, b2, D_out):
    N = nd.shape[0]

    def body(ap_ref, nd_ref, b2_ref, o_ref):
        o_ref[...] = (
            ap_ref[0, :N, :D_out] + ap_ref[1, :N, :D_out]
        ) * nd_ref[...] + b2_ref[...]

    return pl.pallas_call(
        body,
        out_shape=jax.ShapeDtypeStruct((N, D_out), jnp.float32),
    )(aggp, nd, b2)


def kernel(features, edge_index, W1, b1, W2, b2):
    N, D_in = features.shape
    E = edge_index.shape[1]
    D_hid = W1.shape[1]
    D_out = W2.shape[1]

    src = edge_index[0].astype(jnp.int32)
    dst = edge_index[1].astype(jnp.int32)

    K = 40
    NP = ((N + NS * L - 1) // (NS * L)) * (NS * L)

    degp = _make_degree_kernel(E, NP, K)(src, dst)
    deg_out = degp[0, 0, :N] + degp[1, 0, :N]
    deg_in = degp[0, 1, :N] + degp[1, 1, :N]
    ns = lax.rsqrt(jnp.clip(deg_out, 1.0))[:, None]
    nd = lax.rsqrt(jnp.clip(deg_in, 1.0))[:, None]

    h1s = _tc_mm_scale(features, W1, ns)
    aggp1 = _make_agg_kernel(NP, D_hid, E, K)(h1s, src, dst)
    D2 = 128
    W2p = jnp.pad(W2, ((0, 0), (0, D2 - D_out)))
    h2s = _tc_layer_mid(aggp1, nd, b1, W2p, ns)
    aggp2 = _make_agg_kernel(NP, D2, E, K)(h2s, src, dst)
    return _tc_final(aggp2, nd, b2, D_out)


# speedup vs baseline: 1.5599x; 1.5599x over previous
import functools

import jax
import jax.numpy as jnp
from jax import lax
from jax.experimental import pallas as pl
from jax.experimental.pallas import tpu as pltpu
from jax.experimental.pallas import tpu_sc as plsc

NC = 2
NS = 16
L = 16
NW = NC * NS

_mesh = lambda: plsc.VectorSubcoreMesh(
    core_axis_name="c", subcore_axis_name="s", num_cores=NC, num_subcores=NS
)


def _zero_vmem_2d(ref, rows, width):
    z = jnp.zeros((L,), jnp.float32)

    def body(i, _):
        for j in range(width // L):
            ref[i, pl.ds(j * L, L)] = z
        return 0

    lax.fori_loop(0, rows, body, 0)


def _zero_vmem_1d(ref, n):
    z = jnp.zeros((L,), jnp.float32)

    def body(i, _):
        ref[pl.ds(i * L, L)] = z
        return 0

    lax.fori_loop(0, n // L, body, 0)


def _make_degree_kernel(E, NP, K):
    e_per_tile = E // NW
    n_chunks = e_per_tile // K
    npt = NP // NS

    @functools.partial(
        pl.kernel,
        out_type=jax.ShapeDtypeStruct((NC, 2, NP), jnp.float32),
        mesh=_mesh(),
        scratch_types=[
            pltpu.VMEM_SHARED((NP,), jnp.float32),
            pltpu.VMEM_SHARED((NP,), jnp.float32),
            pltpu.VMEM((K,), jnp.int32),
            pltpu.VMEM((K,), jnp.int32),
            pltpu.VMEM((K,), jnp.float32),
            pltpu.VMEM((npt,), jnp.float32),
        ],
    )
    def deg_kernel(src_hbm, dst_hbm, out_hbm, deg_o, deg_i, sidx, didx, ones, zbuf):
        c = lax.axis_index("c")
        s = lax.axis_index("s")
        g = c * NS + s

        one = jnp.ones((L,), jnp.float32)

        def ones_body(i, _):
            ones[pl.ds(i * L, L)] = one
            return 0

        lax.fori_loop(0, K // L, ones_body, 0)
        _zero_vmem_1d(zbuf, npt)

        base_n = s * npt
        pltpu.sync_copy(zbuf, deg_o.at[pl.ds(base_n, npt)])
        pltpu.sync_copy(zbuf, deg_i.at[pl.ds(base_n, npt)])
        plsc.subcore_barrier()

        def chunk(i, _):
            base_e = g * e_per_tile + i * K
            pltpu.sync_copy(src_hbm.at[pl.ds(base_e, K)], sidx)
            pltpu.sync_copy(dst_hbm.at[pl.ds(base_e, K)], didx)
            pltpu.sync_copy(ones, deg_o.at[sidx], add=True)
            pltpu.sync_copy(ones, deg_i.at[didx], add=True)
            return 0

        lax.fori_loop(0, n_chunks, chunk, 0)
        plsc.subcore_barrier()

        pltpu.sync_copy(deg_o.at[pl.ds(base_n, npt)], out_hbm.at[c, 0, pl.ds(base_n, npt)])
        pltpu.sync_copy(deg_i.at[pl.ds(base_n, npt)], out_hbm.at[c, 1, pl.ds(base_n, npt)])

    return deg_kernel


def _make_agg_kernel(NP, D, E, K):
    e_per_tile = E // NW
    n_chunks = e_per_tile // K
    assert n_chunks % 2 == 1
    rpt = NP // NS
    ZR = 64
    assert rpt % ZR == 0 and rpt % 8 == 0

    @functools.partial(
        pl.kernel,
        out_type=jax.ShapeDtypeStruct((NC, NP, D), jnp.float32),
        mesh=_mesh(),
        scratch_types=[
            pltpu.VMEM_SHARED((NP, D), jnp.float32),
            pltpu.VMEM((2, K), jnp.int32),
            pltpu.VMEM((2, K), jnp.int32),
            pltpu.VMEM((2, K, D), jnp.float32),
            pltpu.VMEM((ZR, D), jnp.float32),
            pltpu.SemaphoreType.DMA,
        ],
    )
    def agg_kernel(h_hbm, src_hbm, dst_hbm, out_hbm, agg, sidx, didx, rows, zbuf,
                   gsem):
        c = lax.axis_index("c")
        s = lax.axis_index("s")
        g = c * NS + s
        e0 = g * e_per_tile

        _zero_vmem_2d(zbuf, ZR, D)
        row0 = s * rpt
        for r in range(rpt // ZR):
            pltpu.sync_copy(zbuf, agg.at[pl.ds(row0 + r * ZR, ZR)])
        plsc.subcore_barrier()

        def fetch(i, slot):
            pltpu.sync_copy(src_hbm.at[pl.ds(e0 + i * K, K)], sidx.at[slot])
            pltpu.sync_copy(dst_hbm.at[pl.ds(e0 + i * K, K)], didx.at[slot])
            pltpu.async_copy(h_hbm.at[sidx.at[slot]], rows.at[slot], gsem)

        def drain_scatter(slot):
            pltpu.make_async_copy(h_hbm.at[sidx.at[slot]], rows.at[slot],
                                  gsem).wait()
            pltpu.sync_copy(rows.at[slot], agg.at[didx.at[slot]], add=True)

        fetch(0, 0)

        def block(j, _):
            fetch(2 * j + 1, 1)
            drain_scatter(0)
            fetch(2 * j + 2, 0)
            drain_scatter(1)
            return 0

        lax.fori_loop(0, (n_chunks - 1) // 2, block, 0)
        drain_scatter(0)
        plsc.subcore_barrier()

        pltpu.sync_copy(agg.at[pl.ds(row0, rpt)], out_hbm.at[c, pl.ds(row0, rpt)])

    return agg_kernel


def _tc_mm_scale(x, w, ns):
    def body(x_ref, w_ref, ns_ref, o_ref):
        o_ref[...] = (
            jnp.dot(x_ref[...], w_ref[...], preferred_element_type=jnp.float32)
            * ns_ref[...]
        )

    return pl.pallas_call(
        body,
        out_shape=jax.ShapeDtypeStruct((x.shape[0], w.shape[1]), jnp.float32),
    )(x, w, ns)


def _tc_layer_mid(aggp, nd, b1, w, ns):
    N = nd.shape[0]

    def body(ap_ref, nd_ref, b1_ref, w_ref, ns_ref, o_ref):
        a = ap_ref[0, :N] + ap_ref[1, :N]
        h = jnp.maximum(a * nd_ref[...] + b1_ref[...], 0.0)
        o_ref[...] = (
            jnp.dot(h, w_ref[...], preferred_element_type=jnp.float32) * ns_ref[...]
        )

    return pl.pallas_call(
        body,
        out_shape=jax.ShapeDtypeStruct((N, w.shape[1]), jnp.float32),
    )(aggp, nd, b1, w, ns)


def _tc_final(aggp, nd, b2, D_out):
    N = nd.shape[0]

    def body(ap_ref, nd_ref, b2_ref, o_ref):
        o_ref[...] = (
            ap_ref[0, :N, :D_out] + ap_ref[1, :N, :D_out]
        ) * nd_ref[...] + b2_ref[...]

    return pl.pallas_call(
        body,
        out_shape=jax.ShapeDtypeStruct((N, D_out), jnp.float32),
    )(aggp, nd, b2)


def kernel(features, edge_index, W1, b1, W2, b2):
    N, D_in = features.shape
    E = edge_index.shape[1]
    D_hid = W1.shape[1]
    D_out = W2.shape[1]

    src = edge_index[0].astype(jnp.int32)
    dst = edge_index[1].astype(jnp.int32)

    K = 80
    NP = ((N + NS * L - 1) // (NS * L)) * (NS * L)

    degp = _make_degree_kernel(E, NP, K)(src, dst)
    deg_out = degp[0, 0, :N] + degp[1, 0, :N]
    deg_in = degp[0, 1, :N] + degp[1, 1, :N]
    ns = lax.rsqrt(jnp.clip(deg_out, 1.0))[:, None]
    nd = lax.rsqrt(jnp.clip(deg_in, 1.0))[:, None]

    h1s = _tc_mm_scale(features, W1, ns)
    aggp1 = _make_agg_kernel(NP, D_hid, E, K)(h1s, src, dst)
    D2 = 128
    W2p = jnp.pad(W2, ((0, 0), (0, D2 - D_out)))
    h2s = _tc_layer_mid(aggp1, nd, b1, W2p, ns)
    aggp2 = _make_agg_kernel(NP, D2, E, K)(h2s, src, dst)
    return _tc_final(aggp2, nd, b2, D_out)


# speedup vs baseline: 1.5779x; 1.0115x over previous
import functools

import jax
import jax.numpy as jnp
from jax import lax
from jax.experimental import pallas as pl
from jax.experimental.pallas import tpu as pltpu
from jax.experimental.pallas import tpu_sc as plsc

NC = 2
NS = 16
L = 16
NW = NC * NS

_mesh = lambda: plsc.VectorSubcoreMesh(
    core_axis_name="c", subcore_axis_name="s", num_cores=NC, num_subcores=NS
)


def _zero_vmem_2d(ref, rows, width):
    z = jnp.zeros((L,), jnp.float32)

    def body(i, _):
        for j in range(width // L):
            ref[i, pl.ds(j * L, L)] = z
        return 0

    lax.fori_loop(0, rows, body, 0)


def _zero_vmem_1d(ref, n):
    z = jnp.zeros((L,), jnp.float32)

    def body(i, _):
        ref[pl.ds(i * L, L)] = z
        return 0

    lax.fori_loop(0, n // L, body, 0)


def _make_degree_kernel(E, NP, K):
    e_per_tile = E // NW
    n_chunks = e_per_tile // K
    npt = NP // NS

    @functools.partial(
        pl.kernel,
        out_type=jax.ShapeDtypeStruct((NC, 2, NP), jnp.float32),
        mesh=_mesh(),
        compiler_params=pltpu.CompilerParams(needs_layout_passes=False),
        scratch_types=[
            pltpu.VMEM_SHARED((NS, NP), jnp.float32),
            pltpu.VMEM_SHARED((NS, NP), jnp.float32),
            pltpu.VMEM((NP,), jnp.float32),
            pltpu.VMEM((NP,), jnp.float32),
            pltpu.VMEM((K,), jnp.int32),
            pltpu.VMEM((K,), jnp.int32),
            pltpu.VMEM((NS, npt), jnp.float32),
            pltpu.VMEM((npt,), jnp.float32),
        ],
    )
    def deg_kernel(src_hbm, dst_hbm, out_hbm, stage_o, stage_i, hist_o, hist_i,
                   sidx, didx, comb, res):
        c = lax.axis_index("c")
        s = lax.axis_index("s")
        g = c * NS + s

        _zero_vmem_1d(hist_o, NP)
        _zero_vmem_1d(hist_i, NP)
        one = jnp.ones((L,), jnp.float32)

        def chunk(i, _):
            base_e = g * e_per_tile + i * K
            pltpu.sync_copy(src_hbm.at[pl.ds(base_e, K)], sidx)
            pltpu.sync_copy(dst_hbm.at[pl.ds(base_e, K)], didx)
            for u in range(K // L):
                plsc.addupdate_scatter(hist_o, [sidx[pl.ds(u * L, L)]], one)
                plsc.addupdate_scatter(hist_i, [didx[pl.ds(u * L, L)]], one)
            return 0

        lax.fori_loop(0, n_chunks, chunk, 0)

        pltpu.sync_copy(hist_o, stage_o.at[s])
        pltpu.sync_copy(hist_i, stage_i.at[s])
        plsc.subcore_barrier()
        base_n = s * npt
        for a, stage in ((0, stage_o), (1, stage_i)):
            pltpu.sync_copy(stage.at[:, pl.ds(base_n, npt)], comb)

            def red(j, _):
                acc = comb[0, pl.ds(j * L, L)]
                for i in range(1, NS):
                    acc = acc + comb[i, pl.ds(j * L, L)]
                res[pl.ds(j * L, L)] = acc
                return 0

            lax.fori_loop(0, npt // L, red, 0)
            pltpu.sync_copy(res, out_hbm.at[c, a, pl.ds(base_n, npt)])

    return deg_kernel


def _make_agg_kernel(NP, D, E, K):
    e_per_tile = E // NW
    n_chunks = e_per_tile // K
    assert n_chunks % 2 == 1
    rpt = NP // NS
    ZR = 64
    assert rpt % ZR == 0 and rpt % 8 == 0

    @functools.partial(
        pl.kernel,
        out_type=jax.ShapeDtypeStruct((NC, NP, D), jnp.float32),
        mesh=_mesh(),
        scratch_types=[
            pltpu.VMEM_SHARED((NP, D), jnp.float32),
            pltpu.VMEM((2, K), jnp.int32),
            pltpu.VMEM((2, K), jnp.int32),
            pltpu.VMEM((2, K, D), jnp.float32),
            pltpu.VMEM((ZR, D), jnp.float32),
            pltpu.SemaphoreType.DMA,
        ],
    )
    def agg_kernel(h_hbm, src_hbm, dst_hbm, out_hbm, agg, sidx, didx, rows, zbuf,
                   gsem):
        c = lax.axis_index("c")
        s = lax.axis_index("s")
        g = c * NS + s
        e0 = g * e_per_tile

        _zero_vmem_2d(zbuf, ZR, D)
        row0 = s * rpt
        for r in range(rpt // ZR):
            pltpu.sync_copy(zbuf, agg.at[pl.ds(row0 + r * ZR, ZR)])
        plsc.subcore_barrier()

        def fetch(i, slot):
            pltpu.sync_copy(src_hbm.at[pl.ds(e0 + i * K, K)], sidx.at[slot])
            pltpu.sync_copy(dst_hbm.at[pl.ds(e0 + i * K, K)], didx.at[slot])
            pltpu.async_copy(h_hbm.at[sidx.at[slot]], rows.at[slot], gsem)

        def drain_scatter(slot):
            pltpu.make_async_copy(h_hbm.at[sidx.at[slot]], rows.at[slot],
                                  gsem).wait()
            pltpu.sync_copy(rows.at[slot], agg.at[didx.at[slot]], add=True)

        fetch(0, 0)

        def block(j, _):
            fetch(2 * j + 1, 1)
            drain_scatter(0)
            fetch(2 * j + 2, 0)
            drain_scatter(1)
            return 0

        lax.fori_loop(0, (n_chunks - 1) // 2, block, 0)
        drain_scatter(0)
        plsc.subcore_barrier()

        pltpu.sync_copy(agg.at[pl.ds(row0, rpt)], out_hbm.at[c, pl.ds(row0, rpt)])

    return agg_kernel


def _tc_mm_scale(x, w, ns):
    def body(x_ref, w_ref, ns_ref, o_ref):
        o_ref[...] = (
            jnp.dot(x_ref[...], w_ref[...], preferred_element_type=jnp.float32)
            * ns_ref[...]
        )

    return pl.pallas_call(
        body,
        out_shape=jax.ShapeDtypeStruct((x.shape[0], w.shape[1]), jnp.float32),
    )(x, w, ns)


def _tc_layer_mid(aggp, nd, b1, w, ns):
    N = nd.shape[0]

    def body(ap_ref, nd_ref, b1_ref, w_ref, ns_ref, o_ref):
        a = ap_ref[0, :N] + ap_ref[1, :N]
        h = jnp.maximum(a * nd_ref[...] + b1_ref[...], 0.0)
        o_ref[...] = (
            jnp.dot(h, w_ref[...], preferred_element_type=jnp.float32) * ns_ref[...]
        )

    return pl.pallas_call(
        body,
        out_shape=jax.ShapeDtypeStruct((N, w.shape[1]), jnp.float32),
    )(aggp, nd, b1, w, ns)


def _tc_final(aggp, nd, b2, D_out):
    N = nd.shape[0]

    def body(ap_ref, nd_ref, b2_ref, o_ref):
        o_ref[...] = (
            ap_ref[0, :N, :D_out] + ap_ref[1, :N, :D_out]
        ) * nd_ref[...] + b2_ref[...]

    return pl.pallas_call(
        body,
        out_shape=jax.ShapeDtypeStruct((N, D_out), jnp.float32),
    )(aggp, nd, b2)


def kernel(features, edge_index, W1, b1, W2, b2):
    N, D_in = features.shape
    E = edge_index.shape[1]
    D_hid = W1.shape[1]
    D_out = W2.shape[1]

    src = edge_index[0].astype(jnp.int32)
    dst = edge_index[1].astype(jnp.int32)

    K = 80
    NP = ((N + NS * L - 1) // (NS * L)) * (NS * L)

    degp = _make_degree_kernel(E, NP, K)(src, dst)
    deg_out = degp[0, 0, :N] + degp[1, 0, :N]
    deg_in = degp[0, 1, :N] + degp[1, 1, :N]
    ns = lax.rsqrt(jnp.clip(deg_out, 1.0))[:, None]
    nd = lax.rsqrt(jnp.clip(deg_in, 1.0))[:, None]

    h1s = _tc_mm_scale(features, W1, ns)
    aggp1 = _make_agg_kernel(NP, D_hid, E, K)(h1s, src, dst)
    D2 = 128
    W2p = jnp.pad(W2, ((0, 0), (0, D2 - D_out)))
    h2s = _tc_layer_mid(aggp1, nd, b1, W2p, ns)
    aggp2 = _make_agg_kernel(NP, D2, E, K)(h2s, src, dst)
    return _tc_final(aggp2, nd, b2, D_out)
